# Initial kernel scaffold; baseline (speedup 1.0000x reference)
import jax
import jax.numpy as jnp
from jax.experimental import pallas as pl


def kernel(position_ids, hidden_states, layernums, KV_cache, Wqkv_w, q_ln_w, q_ln_b, k_ln_w, k_ln_b, out_w):
    raise NotImplementedError("write your pallas kernel here")


# speedup vs baseline: 1.0078x; 1.0078x over previous
import math

import jax
import jax.numpy as jnp
import numpy as np
from jax.experimental import pallas as pl
from jax.experimental.pallas import tpu as pltpu

S = 2048
D_MODEL = 2048
N_HEADS = 16
HEAD_DIM = D_MODEL // N_HEADS
KV_SIZE = D_MODEL
CLIP_QKV = 8.0
ALIBI_BIAS_MAX = 8

M_TILE = 256
QB = 256
SCALE = HEAD_DIM ** -0.5


def _alibi_slopes_np(total_num_heads, alibi_bias_max):
    next_pow2 = 2 ** math.ceil(math.log2(total_num_heads))
    m = np.arange(1, next_pow2 + 1, dtype=np.float32) * (alibi_bias_max / next_pow2)
    slopes = 1.0 / np.power(2.0, m)
    if next_pow2 != total_num_heads:
        slopes = np.concatenate([slopes[1::2], slopes[::2]])[:total_num_heads]
    return slopes.astype(np.float32)


def _qkv_body(h_ref, w_ref, lnw_ref, lnb_ref, o_ref):
    j = pl.program_id(0)
    x = jax.lax.dot_general(
        h_ref[...], w_ref[...], (((1,), (0,)), ((), ())),
        preferred_element_type=jnp.float32)
    x = jnp.clip(x, -CLIP_QKV, CLIP_QKV)
    mu = jnp.mean(x, axis=-1, keepdims=True)
    var = jnp.mean((x - mu) ** 2, axis=-1, keepdims=True)
    ln = (x - mu) * jax.lax.rsqrt(var + 1e-5) * lnw_ref[0] + lnb_ref[0]
    o_ref[...] = jnp.where(j < 2, ln, x)


def _attn_body(slopes_ref, q_ref, k_ref, v_ref, wo_ref, o_ref):
    qb = pl.program_id(0)
    h = pl.program_id(1)
    q = q_ref[...] * SCALE
    s = jax.lax.dot_general(
        q, k_ref[...], (((1,), (1,)), ((), ())),
        preferred_element_type=jnp.float32)
    rows = qb * QB + jax.lax.broadcasted_iota(jnp.int32, (QB, S), 0)
    cols = jax.lax.broadcasted_iota(jnp.int32, (QB, S), 1)
    dist = (cols - rows).astype(jnp.float32)
    s = s + slopes_ref[h] * dist
    s = jnp.where(dist <= 0.0, s, -jnp.inf)
    m = jnp.max(s, axis=-1, keepdims=True)
    p = jnp.exp(s - m)
    l = jnp.sum(p, axis=-1, keepdims=True)
    ctx = jax.lax.dot_general(
        p, v_ref[...], (((1,), (0,)), ((), ())),
        preferred_element_type=jnp.float32) / l
    contrib = jax.lax.dot_general(
        ctx, wo_ref[...], (((1,), (0,)), ((), ())),
        preferred_element_type=jnp.float32)

    @pl.when(h == 0)
    def _():
        o_ref[...] = contrib

    @pl.when(h > 0)
    def _():
        o_ref[...] += contrib


def kernel(position_ids, hidden_states, layernums, KV_cache, Wqkv_w,
           q_ln_w, q_ln_b, k_ln_w, k_ln_b, out_w):
    del position_ids, layernums, KV_cache
    hs = hidden_states.reshape(S, D_MODEL)
    ln_w = jnp.stack([q_ln_w, k_ln_w, jnp.ones_like(q_ln_w)]).reshape(3, 1, D_MODEL)
    ln_b = jnp.stack([q_ln_b, k_ln_b, jnp.zeros_like(q_ln_b)]).reshape(3, 1, D_MODEL)

    qkv = pl.pallas_call(
        _qkv_body,
        grid=(3, S // M_TILE),
        in_specs=[
            pl.BlockSpec((M_TILE, D_MODEL), lambda j, i: (i, 0)),
            pl.BlockSpec((D_MODEL, D_MODEL), lambda j, i: (0, j)),
            pl.BlockSpec((1, 1, D_MODEL), lambda j, i: (j, 0, 0)),
            pl.BlockSpec((1, 1, D_MODEL), lambda j, i: (j, 0, 0)),
        ],
        out_specs=pl.BlockSpec((M_TILE, D_MODEL), lambda j, i: (i, j)),
        out_shape=jax.ShapeDtypeStruct((S, 3 * D_MODEL), jnp.float32),
    )(hs, Wqkv_w, ln_w, ln_b)

    q = qkv[:, :D_MODEL]
    k = qkv[:, D_MODEL:2 * D_MODEL]
    v = qkv[:, 2 * D_MODEL:]
    slopes = jnp.asarray(_alibi_slopes_np(N_HEADS, ALIBI_BIAS_MAX))

    out = pl.pallas_call(
        _attn_body,
        grid=(S // QB, N_HEADS),
        in_specs=[
            pl.BlockSpec(memory_space=pltpu.SMEM),
            pl.BlockSpec((QB, HEAD_DIM), lambda i, h: (i, h)),
            pl.BlockSpec((S, HEAD_DIM), lambda i, h: (0, h)),
            pl.BlockSpec((S, HEAD_DIM), lambda i, h: (0, h)),
            pl.BlockSpec((HEAD_DIM, D_MODEL), lambda i, h: (h, 0)),
        ],
        out_specs=pl.BlockSpec((QB, D_MODEL), lambda i, h: (i, 0)),
        out_shape=jax.ShapeDtypeStruct((S, D_MODEL), jnp.float32),
    )(slopes, q, k, v, out_w)

    return out.reshape(1, S, D_MODEL)


# speedup vs baseline: 1.0692x; 1.0610x over previous
import math

import jax
import jax.numpy as jnp
import numpy as np
from jax.experimental import pallas as pl
from jax.experimental.pallas import tpu as pltpu

S = 2048
D_MODEL = 2048
N_HEADS = 16
HEAD_DIM = D_MODEL // N_HEADS
KV_SIZE = D_MODEL
CLIP_QKV = 8.0
ALIBI_BIAS_MAX = 8

M_TILE = 256
QB = 256
SCALE = HEAD_DIM ** -0.5


def _alibi_slopes_np(total_num_heads, alibi_bias_max):
    next_pow2 = 2 ** math.ceil(math.log2(total_num_heads))
    m = np.arange(1, next_pow2 + 1, dtype=np.float32) * (alibi_bias_max / next_pow2)
    slopes = 1.0 / np.power(2.0, m)
    if next_pow2 != total_num_heads:
        slopes = np.concatenate([slopes[1::2], slopes[::2]])[:total_num_heads]
    return slopes.astype(np.float32)


def _qkv_body(h_ref, w_ref, lnw_ref, lnb_ref, o_ref):
    j = pl.program_id(0)
    x = jax.lax.dot_general(
        h_ref[...], w_ref[...], (((1,), (0,)), ((), ())),
        preferred_element_type=jnp.float32)
    x = jnp.clip(x, -CLIP_QKV, CLIP_QKV)
    mu = jnp.mean(x, axis=-1, keepdims=True)
    var = jnp.mean((x - mu) ** 2, axis=-1, keepdims=True)
    ln = (x - mu) * jax.lax.rsqrt(var + 1e-5) * lnw_ref[0] + lnb_ref[0]
    o_ref[...] = jnp.where(j < 2, ln, x).astype(jnp.bfloat16)


def _attn_body(slopes_ref, q_ref, k_ref, v_ref, wo_ref, o_ref):
    qb = pl.program_id(0)
    h = pl.program_id(1)
    s = jax.lax.dot_general(
        q_ref[...], k_ref[...], (((1,), (1,)), ((), ())),
        preferred_element_type=jnp.float32) * SCALE
    rows = qb * QB + jax.lax.broadcasted_iota(jnp.int32, (QB, S), 0)
    cols = jax.lax.broadcasted_iota(jnp.int32, (QB, S), 1)
    dist = (cols - rows).astype(jnp.float32)
    s = s + slopes_ref[h] * dist
    s = jnp.where(dist <= 0.0, s, -jnp.inf)
    m = jnp.max(s, axis=-1, keepdims=True)
    p = jnp.exp(s - m)
    l = jnp.sum(p, axis=-1, keepdims=True)
    ctx = jax.lax.dot_general(
        p.astype(jnp.bfloat16), v_ref[...], (((1,), (0,)), ((), ())),
        preferred_element_type=jnp.float32) / l
    contrib = jax.lax.dot_general(
        ctx.astype(jnp.bfloat16), wo_ref[...], (((1,), (0,)), ((), ())),
        preferred_element_type=jnp.float32)

    @pl.when(h == 0)
    def _():
        o_ref[...] = contrib

    @pl.when(h > 0)
    def _():
        o_ref[...] += contrib


def kernel(position_ids, hidden_states, layernums, KV_cache, Wqkv_w,
           q_ln_w, q_ln_b, k_ln_w, k_ln_b, out_w):
    del position_ids, layernums, KV_cache
    hs = hidden_states.reshape(S, D_MODEL).astype(jnp.bfloat16)
    w_qkv = Wqkv_w.astype(jnp.bfloat16)
    w_out = out_w.astype(jnp.bfloat16)
    ln_w = jnp.stack([q_ln_w, k_ln_w, jnp.ones_like(q_ln_w)]).reshape(3, 1, D_MODEL)
    ln_b = jnp.stack([q_ln_b, k_ln_b, jnp.zeros_like(q_ln_b)]).reshape(3, 1, D_MODEL)

    qkv = pl.pallas_call(
        _qkv_body,
        grid=(3, S // M_TILE),
        in_specs=[
            pl.BlockSpec((M_TILE, D_MODEL), lambda j, i: (i, 0)),
            pl.BlockSpec((D_MODEL, D_MODEL), lambda j, i: (0, j)),
            pl.BlockSpec((1, 1, D_MODEL), lambda j, i: (j, 0, 0)),
            pl.BlockSpec((1, 1, D_MODEL), lambda j, i: (j, 0, 0)),
        ],
        out_specs=pl.BlockSpec((M_TILE, D_MODEL), lambda j, i: (i, j)),
        out_shape=jax.ShapeDtypeStruct((S, 3 * D_MODEL), jnp.bfloat16),
    )(hs, w_qkv, ln_w, ln_b)

    q = qkv[:, :D_MODEL]
    k = qkv[:, D_MODEL:2 * D_MODEL]
    v = qkv[:, 2 * D_MODEL:]
    slopes = jnp.asarray(_alibi_slopes_np(N_HEADS, ALIBI_BIAS_MAX))

    out = pl.pallas_call(
        _attn_body,
        grid=(S // QB, N_HEADS),
        in_specs=[
            pl.BlockSpec(memory_space=pltpu.SMEM),
            pl.BlockSpec((QB, HEAD_DIM), lambda i, h: (i, h)),
            pl.BlockSpec((S, HEAD_DIM), lambda i, h: (0, h)),
            pl.BlockSpec((S, HEAD_DIM), lambda i, h: (0, h)),
            pl.BlockSpec((HEAD_DIM, D_MODEL), lambda i, h: (h, 0)),
        ],
        out_specs=pl.BlockSpec((QB, D_MODEL), lambda i, h: (i, 0)),
        out_shape=jax.ShapeDtypeStruct((S, D_MODEL), jnp.float32),
    )(slopes, q, k, v, w_out)

    return out.reshape(1, S, D_MODEL)
